# baseline (speedup 1.0000x reference)
import jax
import jax.numpy as jnp
from jax.experimental import pallas as pl
from jax.experimental.pallas import tpu as pltpu

_B, _S, _H, _NH, _L, _E, _TOPK, _V, _FF, _CAP = 1, 2048, 768, 12, 2, 8, 2, 50000, 3072, 1024
_T = _B * _S
_DH = _H // _NH
_EPAD = 128
_NEG = -1e30
_F32 = jnp.float32


def _ln(x, g, b):
    m = jnp.mean(x, axis=-1, keepdims=True)
    d = x - m
    v = jnp.mean(d * d, axis=-1, keepdims=True)
    return d * jax.lax.rsqrt(v + 1e-5) * g + b


def _gelu(x):
    return 0.5 * x * (1.0 + jax.lax.erf(x * 0.7071067811865476))


def _qkv_body(x_ref, w_ref, b_ref, o_ref):
    o_ref[...] = jax.lax.dot_general(
        x_ref[...].astype(jnp.bfloat16), w_ref[...].astype(jnp.bfloat16),
        (((1,), (1,)), ((), ())), preferred_element_type=_F32) + b_ref[...]


def _qkv(x, w, b):
    BM, BN = 512, 768
    return pl.pallas_call(
        _qkv_body,
        grid=(3 * _H // BN, _T // BM),
        in_specs=[
            pl.BlockSpec((BM, _H), lambda n, m: (m, 0)),
            pl.BlockSpec((BN, _H), lambda n, m: (n, 0)),
            pl.BlockSpec((1, BN), lambda n, m: (0, n)),
        ],
        out_specs=pl.BlockSpec((BM, BN), lambda n, m: (m, n)),
        out_shape=jax.ShapeDtypeStruct((_T, 3 * _H), _F32),
    )(x, w, b.reshape(1, -1))


_BQ = 1024


def _mega_body(qkv_ref, wo_ref, bo_ref, res_ref, g_ref, b_ref, rw_ref, rb_ref,
               x1_ref, meta_ref, acc_ref, carry_ref):
    m = pl.program_id(0)
    base = m * _BQ
    for h in range(_NH):
        q = qkv_ref[pl.ds(base, _BQ), pl.ds(h * _DH, _DH)]
        k = qkv_ref[:, pl.ds(_H + h * _DH, _DH)]
        v = qkv_ref[:, pl.ds(2 * _H + h * _DH, _DH)]
        s = jax.lax.dot_general(q.astype(jnp.bfloat16), k.astype(jnp.bfloat16),
                                (((1,), (1,)), ((), ())),
                                preferred_element_type=_F32) * 0.125
        mx = jnp.max(s, axis=-1, keepdims=True)
        p = jnp.exp(s - mx)
        p = p / jnp.sum(p, axis=-1, keepdims=True)
        acc_ref[:, pl.ds(h * _DH, _DH)] = jnp.dot(
            p.astype(jnp.bfloat16), v.astype(jnp.bfloat16),
            preferred_element_type=_F32)
    y = jax.lax.dot_general(acc_ref[...].astype(jnp.bfloat16),
                            wo_ref[...].astype(jnp.bfloat16),
                            (((1,), (1,)), ((), ())),
                            preferred_element_type=_F32) + bo_ref[...]
    x1 = _ln(res_ref[...] + y, g_ref[...], b_ref[...])
    x1_ref[...] = x1

    @pl.when(m == 0)
    def _():
        carry_ref[...] = jnp.zeros_like(carry_ref)

    logits = jnp.dot(x1, rw_ref[...], preferred_element_type=_F32) + rb_ref[...]
    lane = jax.lax.broadcasted_iota(jnp.int32, (_BQ, _EPAD), 1)
    m1 = jnp.max(logits, axis=1, keepdims=True)
    a1 = jnp.min(jnp.where(logits == m1, lane, _EPAD), axis=1, keepdims=True)
    masked = jnp.where(lane == a1, _NEG, logits)
    m2 = jnp.max(masked, axis=1, keepdims=True)
    a2 = jnp.min(jnp.where(masked == m2, lane, _EPAD), axis=1, keepdims=True)
    eexp = jnp.exp(m2 - m1)
    w1 = 1.0 / (1.0 + eexp)
    w2 = eexp * w1
    oh1 = (lane == a1).astype(_F32)
    oh2 = (lane == a2).astype(_F32)
    ohm = oh1 + oh2
    row_i = jax.lax.broadcasted_iota(jnp.int32, (_BQ, _BQ), 0)
    col_i = jax.lax.broadcasted_iota(jnp.int32, (_BQ, _BQ), 1)
    tril = (row_i > col_i).astype(_F32)
    rank = jnp.dot(tril, ohm, preferred_element_type=_F32) + carry_ref[...]
    r1 = jnp.sum(rank * oh1, axis=1, keepdims=True)
    r2 = jnp.sum(rank * oh2, axis=1, keepdims=True)
    carry_ref[...] = carry_ref[...] + jnp.sum(ohm, axis=0, keepdims=True)
    a1f = a1.astype(_F32)
    a2f = a2.astype(_F32)
    meta = (jnp.where(lane == 0, a1f, 0.0) + jnp.where(lane == 1, a2f, 0.0)
            + jnp.where(lane == 2, w1, 0.0) + jnp.where(lane == 3, w2, 0.0)
            + jnp.where(lane == 4, r1, 0.0) + jnp.where(lane == 5, r2, 0.0))
    meta_ref[...] = meta


def _mega(qkv, wo, bo, res, g, b, rw_pad, rb_pad):
    return pl.pallas_call(
        _mega_body,
        grid=(_T // _BQ,),
        in_specs=[
            pl.BlockSpec((_T, 3 * _H), lambda m: (0, 0)),
            pl.BlockSpec((_H, _H), lambda m: (0, 0)),
            pl.BlockSpec((1, _H), lambda m: (0, 0)),
            pl.BlockSpec((_BQ, _H), lambda m: (m, 0)),
            pl.BlockSpec((1, _H), lambda m: (0, 0)),
            pl.BlockSpec((1, _H), lambda m: (0, 0)),
            pl.BlockSpec((_H, _EPAD), lambda m: (0, 0)),
            pl.BlockSpec((1, _EPAD), lambda m: (0, 0)),
        ],
        out_specs=[
            pl.BlockSpec((_BQ, _H), lambda m: (m, 0)),
            pl.BlockSpec((_BQ, _EPAD), lambda m: (m, 0)),
        ],
        out_shape=[
            jax.ShapeDtypeStruct((_T, _H), _F32),
            jax.ShapeDtypeStruct((_T, _EPAD), _F32),
        ],
        scratch_shapes=[pltpu.VMEM((_BQ, _H), _F32), pltpu.VMEM((1, _EPAD), _F32)],
    )(qkv, wo, bo.reshape(1, -1), res, g.reshape(1, -1), b.reshape(1, -1),
      rw_pad, rb_pad.reshape(1, -1))


_NFF = 4
_BF = _FF // _NFF
_NCH = 8
_MCH = _CAP // _NCH


def _moe_body(cnt_ref, tok_ref, eg_ref, eb_ref, w1_ref, b1_ref, w2_ref, b2_ref,
              out_ref, ln_ref):
    e = pl.program_id(0)
    f = pl.program_id(1)
    cnt = cnt_ref[e]

    @pl.when(f == 0)
    def _():
        tok = tok_ref[...]
        ln_ref[...] = _ln(tok, eg_ref[0], eb_ref[0])
        out_ref[...] = tok + b2_ref[0]

    for c in range(_NCH):
        @pl.when(cnt > c * _MCH)
        def _():
            h = _gelu(jnp.dot(ln_ref[c * _MCH:(c + 1) * _MCH, :].astype(jnp.bfloat16),
                              w1_ref[0, 0].astype(jnp.bfloat16),
                              preferred_element_type=_F32) + b1_ref[0])
            out_ref[c * _MCH:(c + 1) * _MCH, :] += jnp.dot(
                h.astype(jnp.bfloat16), w2_ref[0, 0].astype(jnp.bfloat16),
                preferred_element_type=_F32)


def _moe_ff(counts, tok_buf, eg, eb, W1, b1, W2, b2, l):
    grid_spec = pltpu.PrefetchScalarGridSpec(
        num_scalar_prefetch=1,
        grid=(_E, _NFF),
        in_specs=[
            pl.BlockSpec((_CAP, _H), lambda e, f, c: (e, 0)),
            pl.BlockSpec((1, 1, _H), lambda e, f, c: (e, 0, 0)),
            pl.BlockSpec((1, 1, _H), lambda e, f, c: (e, 0, 0)),
            pl.BlockSpec((1, 1, _H, _BF), lambda e, f, c: (l, e, 0, f)),
            pl.BlockSpec((1, 1, _BF), lambda e, f, c: (e, 0, f)),
            pl.BlockSpec((1, 1, _BF, _H), lambda e, f, c: (l, e, f, 0)),
            pl.BlockSpec((1, 1, _H), lambda e, f, c: (e, 0, 0)),
        ],
        out_specs=pl.BlockSpec((_CAP, _H), lambda e, f, c: (e, 0)),
        scratch_shapes=[pltpu.VMEM((_CAP, _H), _F32)],
    )
    return pl.pallas_call(
        _moe_body,
        grid_spec=grid_spec,
        out_shape=jax.ShapeDtypeStruct((_E * _CAP, _H), _F32),
    )(counts, tok_buf, eg.reshape(_E, 1, _H), eb.reshape(_E, 1, _H),
      W1, b1.reshape(_E, 1, _FF), W2, b2.reshape(_E, 1, _H))


def _comb_body(x_ref, g0_ref, g1_ref, wv_ref, g_ref, b_ref, out_ref):
    wv = wv_ref[...]
    x = x_ref[...] + g0_ref[...] * wv[:, 0:1] + g1_ref[...] * wv[:, 1:2]
    out_ref[...] = _ln(x, g_ref[...], b_ref[...])


def _comb_ln(x1, g0, g1, wv, g, b):
    BM = 512
    return pl.pallas_call(
        _comb_body,
        grid=(_T // BM,),
        in_specs=[
            pl.BlockSpec((BM, _H), lambda m: (m, 0)),
            pl.BlockSpec((BM, _H), lambda m: (m, 0)),
            pl.BlockSpec((BM, _H), lambda m: (m + _T // 512, 0)),
            pl.BlockSpec((BM, _EPAD), lambda m: (m, 0)),
            pl.BlockSpec((1, _H), lambda m: (0, 0)),
            pl.BlockSpec((1, _H), lambda m: (0, 0)),
        ],
        out_specs=pl.BlockSpec((BM, _H), lambda m: (m, 0)),
        out_shape=jax.ShapeDtypeStruct((_T, _H), _F32),
    )(x1, g0, g1, wv, g.reshape(1, -1), b.reshape(1, -1))


def kernel(input_ids, tok_emb, pos_emb, in_proj_w, in_proj_b, out_w, out_b,
           ln_g, ln_b, e_ln_g, e_ln_b, W1, b1, W2, b2, router_w, router_b):
    ids = input_ids.reshape(_T).astype(jnp.int32)
    x = tok_emb[ids] + pos_emb
    tokids = jnp.arange(_T, dtype=jnp.int32)

    for l in range(_L):
        qkv = _qkv(x, in_proj_w, in_proj_b)
        rw_pad = jnp.concatenate(
            [router_w[l], jnp.zeros((_H, _EPAD - _E), _F32)], axis=1)
        rb_pad = jnp.concatenate(
            [router_b[l], jnp.full((_EPAD - _E,), _NEG, _F32)])
        x1, meta = _mega(qkv, out_w, out_b, x, ln_g, ln_b, rw_pad, rb_pad)

        a1 = meta[:, 0].astype(jnp.int32)
        a2 = meta[:, 1].astype(jnp.int32)
        w1v = meta[:, 2]
        w2v = meta[:, 3]
        r1 = meta[:, 4].astype(jnp.int32)
        r2 = meta[:, 5].astype(jnp.int32)
        valid1 = r1 < _CAP
        valid2 = r2 < _CAP
        slot1 = a1 * _CAP + r1
        slot2 = a2 * _CAP + r2

        pos = (jnp.zeros((_E * _CAP + 1,), jnp.int32)
               .at[jnp.where(valid1, slot1, _E * _CAP)].set(tokids)
               .at[jnp.where(valid2, slot2, _E * _CAP)].set(tokids))[:_E * _CAP]
        tok_buf = x1[pos]

        counts = (jnp.zeros((_E,), jnp.int32).at[a1].add(1).at[a2].add(1))
        eo = _moe_ff(counts, tok_buf, e_ln_g[l], e_ln_b[l],
                     W1, b1[l], W2, b2[l], l)

        cat_idx = jnp.concatenate([jnp.where(valid1, slot1, 0),
                                   jnp.where(valid2, slot2, 0)])
        cat = eo[cat_idx]
        wv = jnp.concatenate(
            [(w1v * valid1.astype(_F32))[:, None],
             (w2v * valid2.astype(_F32))[:, None],
             jnp.zeros((_T, _EPAD - 2), _F32)], axis=1)
        x = _comb_ln(x1, cat, cat, wv, ln_g, ln_b)

    return x.reshape(_B, _S, _H)


# speedup vs baseline: 1.1721x; 1.1721x over previous
import jax
import jax.numpy as jnp
from jax.experimental import pallas as pl
from jax.experimental.pallas import tpu as pltpu

_B, _S, _H, _NH, _L, _E, _TOPK, _V, _FF, _CAP = 1, 2048, 768, 12, 2, 8, 2, 50000, 3072, 1024
_T = _B * _S
_DH = _H // _NH
_EPAD = 128
_NEG = -1e30
_F32 = jnp.float32


def _ln(x, g, b):
    m = jnp.mean(x, axis=-1, keepdims=True)
    d = x - m
    v = jnp.mean(d * d, axis=-1, keepdims=True)
    return d * jax.lax.rsqrt(v + 1e-5) * g + b


def _gelu(x):
    return 0.5 * x * (1.0 + jax.lax.erf(x * 0.7071067811865476))


def _qkv_body(x_ref, w_ref, b_ref, o_ref):
    o_ref[...] = jax.lax.dot_general(
        x_ref[...].astype(jnp.bfloat16), w_ref[...].astype(jnp.bfloat16),
        (((1,), (1,)), ((), ())), preferred_element_type=_F32) + b_ref[...]


def _qkv(x, w, b):
    BM, BN = 512, 768
    return pl.pallas_call(
        _qkv_body,
        grid=(3 * _H // BN, _T // BM),
        in_specs=[
            pl.BlockSpec((BM, _H), lambda n, m: (m, 0)),
            pl.BlockSpec((BN, _H), lambda n, m: (n, 0)),
            pl.BlockSpec((1, BN), lambda n, m: (0, n)),
        ],
        out_specs=pl.BlockSpec((BM, BN), lambda n, m: (m, n)),
        out_shape=jax.ShapeDtypeStruct((_T, 3 * _H), _F32),
    )(x, w, b.reshape(1, -1))


_BQ = 512


def _mega_body(qkv_ref, wo_ref, bo_ref, res_ref, g_ref, b_ref, rw_ref, rb_ref,
               x1_ref, meta_ref, acc_ref, carry_ref):
    m = pl.program_id(0)
    base = m * _BQ
    for h in range(_NH):
        q = qkv_ref[pl.ds(base, _BQ), pl.ds(h * _DH, _DH)]
        k = qkv_ref[:, pl.ds(_H + h * _DH, _DH)]
        v = qkv_ref[:, pl.ds(2 * _H + h * _DH, _DH)]
        s = jax.lax.dot_general(q.astype(jnp.bfloat16), k.astype(jnp.bfloat16),
                                (((1,), (1,)), ((), ())),
                                preferred_element_type=_F32) * 0.125
        mx = jnp.max(s, axis=-1, keepdims=True)
        p = jnp.exp(s - mx)
        p = p / jnp.sum(p, axis=-1, keepdims=True)
        acc_ref[:, pl.ds(h * _DH, _DH)] = jnp.dot(
            p.astype(jnp.bfloat16), v.astype(jnp.bfloat16),
            preferred_element_type=_F32)
    y = jax.lax.dot_general(acc_ref[...].astype(jnp.bfloat16),
                            wo_ref[...].astype(jnp.bfloat16),
                            (((1,), (1,)), ((), ())),
                            preferred_element_type=_F32) + bo_ref[...]
    x1 = _ln(res_ref[...] + y, g_ref[...], b_ref[...])
    x1_ref[...] = x1

    @pl.when(m == 0)
    def _():
        carry_ref[...] = jnp.zeros_like(carry_ref)

    logits = jnp.dot(x1, rw_ref[...], preferred_element_type=_F32) + rb_ref[...]
    lane = jax.lax.broadcasted_iota(jnp.int32, (_BQ, _EPAD), 1)
    m1 = jnp.max(logits, axis=1, keepdims=True)
    a1 = jnp.min(jnp.where(logits == m1, lane, _EPAD), axis=1, keepdims=True)
    masked = jnp.where(lane == a1, _NEG, logits)
    m2 = jnp.max(masked, axis=1, keepdims=True)
    a2 = jnp.min(jnp.where(masked == m2, lane, _EPAD), axis=1, keepdims=True)
    eexp = jnp.exp(m2 - m1)
    w1 = 1.0 / (1.0 + eexp)
    w2 = eexp * w1
    oh1 = (lane == a1).astype(_F32)
    oh2 = (lane == a2).astype(_F32)
    ohm = oh1 + oh2
    row_i = jax.lax.broadcasted_iota(jnp.int32, (_BQ, _BQ), 0)
    col_i = jax.lax.broadcasted_iota(jnp.int32, (_BQ, _BQ), 1)
    tril = (row_i > col_i).astype(_F32)
    rank = jnp.dot(tril, ohm, preferred_element_type=_F32) + carry_ref[...]
    r1 = jnp.sum(rank * oh1, axis=1, keepdims=True)
    r2 = jnp.sum(rank * oh2, axis=1, keepdims=True)
    carry_ref[...] = carry_ref[...] + jnp.sum(ohm, axis=0, keepdims=True)
    a1f = a1.astype(_F32)
    a2f = a2.astype(_F32)
    meta = (jnp.where(lane == 0, a1f, 0.0) + jnp.where(lane == 1, a2f, 0.0)
            + jnp.where(lane == 2, w1, 0.0) + jnp.where(lane == 3, w2, 0.0)
            + jnp.where(lane == 4, r1, 0.0) + jnp.where(lane == 5, r2, 0.0))
    meta_ref[...] = meta


def _mega(qkv, wo, bo, res, g, b, rw_pad, rb_pad):
    return pl.pallas_call(
        _mega_body,
        grid=(_T // _BQ,),
        in_specs=[
            pl.BlockSpec((_T, 3 * _H), lambda m: (0, 0)),
            pl.BlockSpec((_H, _H), lambda m: (0, 0)),
            pl.BlockSpec((1, _H), lambda m: (0, 0)),
            pl.BlockSpec((_BQ, _H), lambda m: (m, 0)),
            pl.BlockSpec((1, _H), lambda m: (0, 0)),
            pl.BlockSpec((1, _H), lambda m: (0, 0)),
            pl.BlockSpec((_H, _EPAD), lambda m: (0, 0)),
            pl.BlockSpec((1, _EPAD), lambda m: (0, 0)),
        ],
        out_specs=[
            pl.BlockSpec((_BQ, _H), lambda m: (m, 0)),
            pl.BlockSpec((_BQ, _EPAD), lambda m: (m, 0)),
        ],
        out_shape=[
            jax.ShapeDtypeStruct((_T, _H), _F32),
            jax.ShapeDtypeStruct((_T, _EPAD), _F32),
        ],
        scratch_shapes=[pltpu.VMEM((_BQ, _H), _F32), pltpu.VMEM((1, _EPAD), _F32)],
    )(qkv, wo, bo.reshape(1, -1), res, g.reshape(1, -1), b.reshape(1, -1),
      rw_pad, rb_pad.reshape(1, -1))


_NFF = 2
_BF = _FF // _NFF
_NCH = 8
_MCH = _CAP // _NCH


def _moe_body(cnt_ref, tok_ref, eg_ref, eb_ref, w1_ref, b1_ref, w2_ref, b2_ref,
              out_ref, ln_ref):
    e = pl.program_id(0)
    f = pl.program_id(1)
    cnt = cnt_ref[e]

    @pl.when(f == 0)
    def _():
        tok = tok_ref[...]
        ln_ref[...] = _ln(tok, eg_ref[0], eb_ref[0])
        out_ref[...] = tok + b2_ref[0]

    for c in range(_NCH):
        @pl.when(cnt > c * _MCH)
        def _():
            h = _gelu(jnp.dot(ln_ref[c * _MCH:(c + 1) * _MCH, :].astype(jnp.bfloat16),
                              w1_ref[0, 0].astype(jnp.bfloat16),
                              preferred_element_type=_F32) + b1_ref[0])
            out_ref[c * _MCH:(c + 1) * _MCH, :] += jnp.dot(
                h.astype(jnp.bfloat16), w2_ref[0, 0].astype(jnp.bfloat16),
                preferred_element_type=_F32)


def _moe_ff(counts, tok_buf, eg, eb, W1, b1, W2, b2, l):
    grid_spec = pltpu.PrefetchScalarGridSpec(
        num_scalar_prefetch=1,
        grid=(_E, _NFF),
        in_specs=[
            pl.BlockSpec((_CAP, _H), lambda e, f, c: (e, 0)),
            pl.BlockSpec((1, 1, _H), lambda e, f, c: (e, 0, 0)),
            pl.BlockSpec((1, 1, _H), lambda e, f, c: (e, 0, 0)),
            pl.BlockSpec((1, 1, _H, _BF), lambda e, f, c: (l, e, 0, f)),
            pl.BlockSpec((1, 1, _BF), lambda e, f, c: (e, 0, f)),
            pl.BlockSpec((1, 1, _BF, _H), lambda e, f, c: (l, e, f, 0)),
            pl.BlockSpec((1, 1, _H), lambda e, f, c: (e, 0, 0)),
        ],
        out_specs=pl.BlockSpec((_CAP, _H), lambda e, f, c: (e, 0)),
        scratch_shapes=[pltpu.VMEM((_CAP, _H), _F32)],
    )
    return pl.pallas_call(
        _moe_body,
        grid_spec=grid_spec,
        out_shape=jax.ShapeDtypeStruct((_E * _CAP, _H), _F32),
    )(counts, tok_buf, eg.reshape(_E, 1, _H), eb.reshape(_E, 1, _H),
      W1, b1.reshape(_E, 1, _FF), W2, b2.reshape(_E, 1, _H))


def _comb_body(x_ref, g0_ref, g1_ref, wv_ref, g_ref, b_ref, out_ref):
    wv = wv_ref[...]
    x = x_ref[...] + g0_ref[...] * wv[:, 0:1] + g1_ref[...] * wv[:, 1:2]
    out_ref[...] = _ln(x, g_ref[...], b_ref[...])


def _comb_ln(x1, g0, g1, wv, g, b):
    BM = 512
    return pl.pallas_call(
        _comb_body,
        grid=(_T // BM,),
        in_specs=[
            pl.BlockSpec((BM, _H), lambda m: (m, 0)),
            pl.BlockSpec((BM, _H), lambda m: (m, 0)),
            pl.BlockSpec((BM, _H), lambda m: (m + _T // 512, 0)),
            pl.BlockSpec((BM, _EPAD), lambda m: (m, 0)),
            pl.BlockSpec((1, _H), lambda m: (0, 0)),
            pl.BlockSpec((1, _H), lambda m: (0, 0)),
        ],
        out_specs=pl.BlockSpec((BM, _H), lambda m: (m, 0)),
        out_shape=jax.ShapeDtypeStruct((_T, _H), _F32),
    )(x1, g0, g1, wv, g.reshape(1, -1), b.reshape(1, -1))


def kernel(input_ids, tok_emb, pos_emb, in_proj_w, in_proj_b, out_w, out_b,
           ln_g, ln_b, e_ln_g, e_ln_b, W1, b1, W2, b2, router_w, router_b):
    ids = input_ids.reshape(_T).astype(jnp.int32)
    x = tok_emb[ids] + pos_emb
    tokids = jnp.arange(_T, dtype=jnp.int32)

    for l in range(_L):
        qkv = _qkv(x, in_proj_w, in_proj_b)
        rw_pad = jnp.concatenate(
            [router_w[l], jnp.zeros((_H, _EPAD - _E), _F32)], axis=1)
        rb_pad = jnp.concatenate(
            [router_b[l], jnp.full((_EPAD - _E,), _NEG, _F32)])
        x1, meta = _mega(qkv, out_w, out_b, x, ln_g, ln_b, rw_pad, rb_pad)

        a1 = meta[:, 0].astype(jnp.int32)
        a2 = meta[:, 1].astype(jnp.int32)
        w1v = meta[:, 2]
        w2v = meta[:, 3]
        r1 = meta[:, 4].astype(jnp.int32)
        r2 = meta[:, 5].astype(jnp.int32)
        valid1 = r1 < _CAP
        valid2 = r2 < _CAP
        slot1 = a1 * _CAP + r1
        slot2 = a2 * _CAP + r2

        pos = (jnp.zeros((_E * _CAP + 1,), jnp.int32)
               .at[jnp.where(valid1, slot1, _E * _CAP)].set(tokids)
               .at[jnp.where(valid2, slot2, _E * _CAP)].set(tokids))[:_E * _CAP]
        tok_buf = x1[pos]

        counts = (jnp.zeros((_E,), jnp.int32).at[a1].add(1).at[a2].add(1))
        eo = _moe_ff(counts, tok_buf, e_ln_g[l], e_ln_b[l],
                     W1, b1[l], W2, b2[l], l)

        cat_idx = jnp.concatenate([jnp.where(valid1, slot1, 0),
                                   jnp.where(valid2, slot2, 0)])
        cat = eo[cat_idx]
        wv = jnp.concatenate(
            [(w1v * valid1.astype(_F32))[:, None],
             (w2v * valid2.astype(_F32))[:, None],
             jnp.zeros((_T, _EPAD - 2), _F32)], axis=1)
        x = _comb_ln(x1, cat, cat, wv, ln_g, ln_b)

    return x.reshape(_B, _S, _H)
